# baseline (speedup 1.0000x reference)
import jax
import jax.numpy as jnp
from jax import lax
from jax.experimental import pallas as pl
from jax.experimental.pallas import tpu as pltpu

_TN = 1024
_EPS = 1e-8


def _body(z_ref, c_ref, dist_ref, assign_ref, csq_ref):
    zb = z_ref[...]
    c = c_ref[...]
    k, d = c.shape
    ones_col = jnp.ones((d, 1), jnp.float32)
    ones_row = jnp.ones((1, d), jnp.float32)

    @pl.when(pl.program_id(0) == 0)
    def _init():
        cc = c * c
        csq_ref[...] = lax.dot_general(
            ones_row, cc, (((1,), (1,)), ((), ())),
            precision=lax.Precision.HIGHEST,
            preferred_element_type=jnp.float32)

    c_sq_row = csq_ref[...]
    z_sq = lax.dot_general(zb * zb, ones_col, (((1,), (0,)), ((), ())),
                           preferred_element_type=jnp.float32)
    g = lax.dot_general(zb, -2.0 * c, (((1,), (1,)), ((), ())),
                        preferred_element_type=jnp.float32)
    d2 = g + c_sq_row
    m = jnp.min(d2, axis=1, keepdims=True)
    eq = d2 == m
    idx_row = lax.broadcasted_iota(jnp.int32, (1, k), 1).astype(jnp.float32)
    a_f = jnp.min(jnp.where(eq, idx_row, jnp.float32(k)),
                  axis=1, keepdims=True)
    c_sq_a = jnp.max(jnp.where(eq, c_sq_row, -jnp.inf),
                     axis=1, keepdims=True)
    dot_a = 0.5 * (c_sq_a - m)
    inv_zn = lax.rsqrt(jnp.maximum(z_sq, _EPS * _EPS))
    inv_cn = lax.rsqrt(jnp.maximum(c_sq_a, _EPS * _EPS))
    dist_ref[...] = 1.0 - dot_a * (inv_zn * inv_cn)
    assign_ref[...] = a_f.astype(jnp.int32)


def kernel(z, centers):
    n, d = z.shape
    k, _ = centers.shape
    grid = (n // _TN,)
    dist, assign = pl.pallas_call(
        _body,
        grid=grid,
        in_specs=[
            pl.BlockSpec((_TN, d), lambda i: (i, 0)),
            pl.BlockSpec((k, d), lambda i: (0, 0)),
        ],
        out_specs=[
            pl.BlockSpec((_TN, 1), lambda i: (i, 0)),
            pl.BlockSpec((_TN, 1), lambda i: (i, 0)),
        ],
        out_shape=[
            jax.ShapeDtypeStruct((n, 1), jnp.float32),
            jax.ShapeDtypeStruct((n, 1), jnp.int32),
        ],
        scratch_shapes=[pltpu.VMEM((1, k), jnp.float32)],
        compiler_params=pltpu.CompilerParams(
            dimension_semantics=("arbitrary",)),
    )(z, centers)
    return dist.reshape(n), assign.reshape(n)


# speedup vs baseline: 1.0367x; 1.0367x over previous
import jax
import jax.numpy as jnp
from jax import lax
from jax.experimental import pallas as pl
from jax.experimental.pallas import tpu as pltpu

_TN = 2048
_EPS = 1e-8


def _body(z_ref, c_ref, dist_ref, assign_ref, csq_ref):
    zb = z_ref[...]
    c = c_ref[...]
    k, d = c.shape
    ones_col = jnp.ones((d, 1), jnp.float32)
    ones_row = jnp.ones((1, d), jnp.float32)

    @pl.when(pl.program_id(0) == 0)
    def _init():
        cc = c * c
        csq_ref[...] = lax.dot_general(
            ones_row, cc, (((1,), (1,)), ((), ())),
            precision=lax.Precision.HIGHEST,
            preferred_element_type=jnp.float32)

    c_sq_row = csq_ref[...]
    z_sq = lax.dot_general(zb * zb, ones_col, (((1,), (0,)), ((), ())),
                           preferred_element_type=jnp.float32)
    g = lax.dot_general(zb, -2.0 * c, (((1,), (1,)), ((), ())),
                        preferred_element_type=jnp.float32)
    d2 = g + c_sq_row
    m = jnp.min(d2, axis=1, keepdims=True)
    eq = d2 == m
    idx_row = lax.broadcasted_iota(jnp.int32, (1, k), 1).astype(jnp.float32)
    a_f = jnp.min(jnp.where(eq, idx_row, jnp.float32(k)),
                  axis=1, keepdims=True)
    c_sq_a = jnp.max(jnp.where(eq, c_sq_row, -jnp.inf),
                     axis=1, keepdims=True)
    dot_a = 0.5 * (c_sq_a - m)
    inv_zn = lax.rsqrt(jnp.maximum(z_sq, _EPS * _EPS))
    inv_cn = lax.rsqrt(jnp.maximum(c_sq_a, _EPS * _EPS))
    dist_ref[...] = 1.0 - dot_a * (inv_zn * inv_cn)
    assign_ref[...] = a_f.astype(jnp.int32)


def kernel(z, centers):
    n, d = z.shape
    k, _ = centers.shape
    grid = (n // _TN,)
    dist, assign = pl.pallas_call(
        _body,
        grid=grid,
        in_specs=[
            pl.BlockSpec((_TN, d), lambda i: (i, 0)),
            pl.BlockSpec((k, d), lambda i: (0, 0)),
        ],
        out_specs=[
            pl.BlockSpec((_TN, 1), lambda i: (i, 0)),
            pl.BlockSpec((_TN, 1), lambda i: (i, 0)),
        ],
        out_shape=[
            jax.ShapeDtypeStruct((n, 1), jnp.float32),
            jax.ShapeDtypeStruct((n, 1), jnp.int32),
        ],
        scratch_shapes=[pltpu.VMEM((1, k), jnp.float32)],
        compiler_params=pltpu.CompilerParams(
            dimension_semantics=("arbitrary",)),
    )(z, centers)
    return dist.reshape(n), assign.reshape(n)


# speedup vs baseline: 1.0407x; 1.0039x over previous
import jax
import jax.numpy as jnp
from jax import lax
from jax.experimental import pallas as pl
from jax.experimental.pallas import tpu as pltpu

_TN = 4096
_EPS = 1e-8


def _body(z_ref, c_ref, dist_ref, assign_ref, csq_ref):
    zb = z_ref[...]
    c = c_ref[...]
    k, d = c.shape
    ones_col = jnp.ones((d, 1), jnp.float32)
    ones_row = jnp.ones((1, d), jnp.float32)

    @pl.when(pl.program_id(0) == 0)
    def _init():
        cc = c * c
        csq_ref[...] = lax.dot_general(
            ones_row, cc, (((1,), (1,)), ((), ())),
            precision=lax.Precision.HIGHEST,
            preferred_element_type=jnp.float32)

    c_sq_row = csq_ref[...]
    z_sq = lax.dot_general(zb * zb, ones_col, (((1,), (0,)), ((), ())),
                           preferred_element_type=jnp.float32)
    g = lax.dot_general(zb, -2.0 * c, (((1,), (1,)), ((), ())),
                        preferred_element_type=jnp.float32)
    d2 = g + c_sq_row
    m = jnp.min(d2, axis=1, keepdims=True)
    eq = d2 == m
    idx_row = lax.broadcasted_iota(jnp.int32, (1, k), 1).astype(jnp.float32)
    a_f = jnp.min(jnp.where(eq, idx_row, jnp.float32(k)),
                  axis=1, keepdims=True)
    c_sq_a = jnp.max(jnp.where(eq, c_sq_row, -jnp.inf),
                     axis=1, keepdims=True)
    dot_a = 0.5 * (c_sq_a - m)
    inv_zn = lax.rsqrt(jnp.maximum(z_sq, _EPS * _EPS))
    inv_cn = lax.rsqrt(jnp.maximum(c_sq_a, _EPS * _EPS))
    dist_ref[...] = 1.0 - dot_a * (inv_zn * inv_cn)
    assign_ref[...] = a_f.astype(jnp.int32)


def kernel(z, centers):
    n, d = z.shape
    k, _ = centers.shape
    grid = (n // _TN,)
    dist, assign = pl.pallas_call(
        _body,
        grid=grid,
        in_specs=[
            pl.BlockSpec((_TN, d), lambda i: (i, 0)),
            pl.BlockSpec((k, d), lambda i: (0, 0)),
        ],
        out_specs=[
            pl.BlockSpec((_TN, 1), lambda i: (i, 0)),
            pl.BlockSpec((_TN, 1), lambda i: (i, 0)),
        ],
        out_shape=[
            jax.ShapeDtypeStruct((n, 1), jnp.float32),
            jax.ShapeDtypeStruct((n, 1), jnp.int32),
        ],
        scratch_shapes=[pltpu.VMEM((1, k), jnp.float32)],
        compiler_params=pltpu.CompilerParams(
            dimension_semantics=("arbitrary",)),
    )(z, centers)
    return dist.reshape(n), assign.reshape(n)
